# Initial kernel scaffold; baseline (speedup 1.0000x reference)
import jax
import jax.numpy as jnp
from jax.experimental import pallas as pl


def kernel(input_ids, word_emb, pos_emb, tok_type_emb, gamma, beta):
    raise NotImplementedError("write your pallas kernel here")


# speedup vs baseline: 3.0163x; 3.0163x over previous
import dataclasses
import functools

import jax
import jax.numpy as jnp
from jax import lax
from jax.experimental import pallas as pl
from jax.experimental.pallas import tpu as pltpu
from jax.experimental.pallas import tpu_sc as plsc

EPS = 1e-12
NC, NS = 2, 16
NW = NC * NS
CHUNK = 128
ROWS_PER_ITER = 4


def _row_layernorm(rows_v, const_v, r, l, gs, bs, hidden):
    nsub = hidden // 16
    xs = []
    for j in range(nsub):
        sl = pl.ds(j * 16, 16)
        xs.append(rows_v[r, sl] + const_v[l, sl])
    s = xs[0]
    q = xs[0] * xs[0]
    for j in range(1, nsub):
        s = s + xs[j]
        q = q + xs[j] * xs[j]
    tot = jnp.sum(s)
    totq = jnp.sum(q)
    inv_h = 1.0 / hidden
    mean = tot * inv_h
    var = totq * inv_h - mean * mean + EPS
    i = lax.bitcast_convert_type(var, jnp.int32)
    i = jnp.int32(0x5F3759DF) - lax.shift_right_logical(i, 1)
    y = lax.bitcast_convert_type(i, jnp.float32)
    half_v = 0.5 * var
    for _ in range(3):
        y = y * (1.5 - half_v * y * y)
    for j in range(nsub):
        sl = pl.ds(j * 16, 16)
        rows_v[r, sl] = (xs[j] - mean) * y * gs[j] + bs[j]


def _make_sc_kernel(T, V, H, L):
    per_w = T // NW
    n_chunks = per_w // CHUNK
    mesh = plsc.VectorSubcoreMesh(core_axis_name="c", subcore_axis_name="s")
    cp = pltpu.CompilerParams()
    if "needs_layout_passes" in pltpu.CompilerParams.__dataclass_fields__:
        cp = dataclasses.replace(cp, needs_layout_passes=False)

    @functools.partial(
        pl.kernel,
        mesh=mesh,
        compiler_params=cp,
        out_type=jax.ShapeDtypeStruct((T, H), jnp.float32),
        scratch_types=[
            pltpu.VMEM((L, H), jnp.float32),
            pltpu.VMEM((H,), jnp.float32),
            pltpu.VMEM((H,), jnp.float32),
            pltpu.VMEM((H,), jnp.float32),
            pltpu.VMEM((CHUNK,), jnp.int32),
            pltpu.VMEM((CHUNK, H), jnp.float32),
            pltpu.SemaphoreType.DMA,
        ],
    )
    def k(ids_hbm, table_hbm, pos_hbm, tok_hbm, gamma_hbm, beta_hbm, out_hbm,
          const_v, tok_v, g_v, b_v, idx_v, rows_v, sem):
        wid = lax.axis_index("s") * NC + lax.axis_index("c")
        base = wid * per_w
        nsub = H // 16

        pltpu.sync_copy(pos_hbm.at[pl.ds(0, L)], const_v)
        pltpu.sync_copy(tok_hbm.at[0], tok_v)
        pltpu.sync_copy(gamma_hbm, g_v)
        pltpu.sync_copy(beta_hbm, b_v)

        @pl.loop(0, L)
        def _(r):
            for j in range(nsub):
                sl = pl.ds(j * 16, 16)
                const_v[r, sl] = const_v[r, sl] + tok_v[sl]

        gs = [g_v[pl.ds(j * 16, 16)] for j in range(nsub)]
        bs = [b_v[pl.ds(j * 16, 16)] for j in range(nsub)]

        @pl.loop(0, n_chunks)
        def _(c):
            start = base + c * CHUNK
            pltpu.sync_copy(ids_hbm.at[pl.ds(start, CHUNK)], idx_v)
            pltpu.async_copy(table_hbm.at[idx_v], rows_v, sem).wait()
            lbase = lax.rem(c * CHUNK, L)

            @pl.loop(0, CHUNK, step=ROWS_PER_ITER)
            def _(r0):
                for u in range(ROWS_PER_ITER):
                    r = r0 + u
                    l = lax.rem(lbase + r, L)
                    _row_layernorm(rows_v, const_v, r, l, gs, bs, H)

            pltpu.sync_copy(rows_v, out_hbm.at[pl.ds(start, CHUNK)])

    return k


def kernel(input_ids, word_emb, pos_emb, tok_type_emb, gamma, beta):
    B, L = input_ids.shape
    V, H = word_emb.shape
    T = B * L
    ids = input_ids.reshape(T).astype(jnp.int32)
    k = _make_sc_kernel(T, V, H, L)
    out = k(ids, word_emb, pos_emb, tok_type_emb, gamma, beta)
    return out.reshape(B, L, H)


# speedup vs baseline: 3.8039x; 1.2611x over previous
import dataclasses
import functools

import jax
import jax.numpy as jnp
from jax import lax
from jax.experimental import pallas as pl
from jax.experimental.pallas import tpu as pltpu
from jax.experimental.pallas import tpu_sc as plsc

EPS = 1e-12
NC, NS = 2, 16
NW = NC * NS
CHUNK = 128
NBUF = 5
LOOKAHEAD = 2
ROWS_PER_ITER = 4


def _row_layernorm(rows_v, const_v, r, l, gs, bs, hidden):
    nsub = hidden // 16
    xs = []
    for j in range(nsub):
        sl = pl.ds(j * 16, 16)
        xs.append(rows_v[r, sl] + const_v[l, sl])
    s = xs[0]
    q = xs[0] * xs[0]
    for j in range(1, nsub):
        s = s + xs[j]
        q = q + xs[j] * xs[j]
    tot = jnp.sum(s)
    totq = jnp.sum(q)
    inv_h = 1.0 / hidden
    mean = tot * inv_h
    var = totq * inv_h - mean * mean + EPS
    i = lax.bitcast_convert_type(var, jnp.int32)
    i = jnp.int32(0x5F3759DF) - lax.shift_right_logical(i, 1)
    y = lax.bitcast_convert_type(i, jnp.float32)
    half_v = 0.5 * var
    for _ in range(3):
        y = y * (1.5 - half_v * y * y)
    my = mean * y
    for j in range(nsub):
        sl = pl.ds(j * 16, 16)
        rows_v[r, sl] = (xs[j] * y - my) * gs[j] + bs[j]


def _make_sc_kernel(T, V, H, L):
    per_w = T // NW
    n_chunks = per_w // CHUNK
    n_groups = n_chunks // NBUF
    assert n_chunks % NBUF == 0
    mesh = plsc.VectorSubcoreMesh(core_axis_name="c", subcore_axis_name="s")
    cp = pltpu.CompilerParams()
    if "needs_layout_passes" in pltpu.CompilerParams.__dataclass_fields__:
        cp = dataclasses.replace(cp, needs_layout_passes=False)

    @functools.partial(
        pl.kernel,
        mesh=mesh,
        compiler_params=cp,
        out_type=jax.ShapeDtypeStruct((T, H), jnp.float32),
        scratch_types=(
            [
                pltpu.VMEM((L, H), jnp.float32),
                pltpu.VMEM((H,), jnp.float32),
                pltpu.VMEM((H,), jnp.float32),
                pltpu.VMEM((H,), jnp.float32),
                pltpu.VMEM((per_w,), jnp.int32),
            ]
            + [pltpu.VMEM((CHUNK, H), jnp.float32) for _ in range(NBUF)]
            + [pltpu.SemaphoreType.DMA for _ in range(2 * NBUF + 1)]
        ),
    )
    def k(ids_hbm, table_hbm, pos_hbm, tok_hbm, gamma_hbm, beta_hbm, out_hbm,
          *scratch):
        const_v, tok_v, g_v, b_v, idx_v = scratch[:5]
        rows = scratch[5:5 + NBUF]
        gsem = scratch[5 + NBUF:5 + 2 * NBUF]
        ssem = scratch[5 + 2 * NBUF:5 + 3 * NBUF]
        isem = scratch[5 + 3 * NBUF]

        wid = lax.axis_index("s") * NC + lax.axis_index("c")
        base = wid * per_w
        nsub = H // 16

        idx_cp = pltpu.async_copy(ids_hbm.at[pl.ds(base, per_w)], idx_v, isem)
        pltpu.sync_copy(pos_hbm.at[pl.ds(0, L)], const_v)
        pltpu.sync_copy(tok_hbm.at[0], tok_v)
        pltpu.sync_copy(gamma_hbm, g_v)
        pltpu.sync_copy(beta_hbm, b_v)

        @pl.loop(0, L)
        def _(r):
            for j in range(nsub):
                sl = pl.ds(j * 16, 16)
                const_v[r, sl] = const_v[r, sl] + tok_v[sl]

        gs = [g_v[pl.ds(j * 16, 16)] for j in range(nsub)]
        bs = [b_v[pl.ds(j * 16, 16)] for j in range(nsub)]

        idx_cp.wait()

        def gather_start(c, b):
            pltpu.async_copy(
                table_hbm.at[idx_v.at[pl.ds(c * CHUNK, CHUNK)]], rows[b],
                gsem[b])

        def gather_wait(c, b):
            pltpu.make_async_copy(
                table_hbm.at[idx_v.at[pl.ds(c * CHUNK, CHUNK)]], rows[b],
                gsem[b]).wait()

        def store_start(c, b):
            pltpu.async_copy(rows[b], out_hbm.at[pl.ds(base + c * CHUNK, CHUNK)],
                             ssem[b])

        def store_wait(c, b):
            pltpu.make_async_copy(
                rows[b], out_hbm.at[pl.ds(base + c * CHUNK, CHUNK)],
                ssem[b]).wait()

        for b in range(LOOKAHEAD):
            gather_start(b, b)

        @pl.loop(0, n_groups)
        def _(g):
            for b in range(NBUF):
                c = g * NBUF + b

                b2 = (b + LOOKAHEAD) % NBUF
                c2 = c + LOOKAHEAD

                @pl.when(jnp.logical_and(c2 >= NBUF, c2 < n_chunks))
                def _():
                    store_wait(c2 - NBUF, b2)

                @pl.when(c2 < n_chunks)
                def _():
                    gather_start(c2, b2)

                gather_wait(c, b)

                lbase = lax.rem(c * CHUNK, L)

                @pl.loop(0, CHUNK, step=ROWS_PER_ITER)
                def _(r0):
                    for u in range(ROWS_PER_ITER):
                        r = r0 + u
                        lr = lax.rem(lbase + r, L)
                        _row_layernorm(rows[b], const_v, r, lr, gs, bs, H)

                store_start(c, b)

        for b in range(NBUF):
            store_wait(n_chunks - NBUF + b, b)

    return k


def kernel(input_ids, word_emb, pos_emb, tok_type_emb, gamma, beta):
    B, L = input_ids.shape
    V, H = word_emb.shape
    T = B * L
    ids = input_ids.reshape(T).astype(jnp.int32)
    k = _make_sc_kernel(T, V, H, L)
    out = k(ids, word_emb, pos_emb, tok_type_emb, gamma, beta)
    return out.reshape(B, L, H)


# speedup vs baseline: 10.0911x; 2.6529x over previous
import dataclasses
import functools

import jax
import jax.numpy as jnp
from jax import lax
from jax.experimental import pallas as pl
from jax.experimental.pallas import tpu as pltpu
from jax.experimental.pallas import tpu_sc as plsc

EPS = 1e-12
NC, NS = 2, 16
NW = NC * NS
CHUNK = 128
NBUF = 2
LOOKAHEAD = 2
ROWS_PER_ITER = 2
POS_ROWS = 16


_GATHER_DNUMS = lax.GatherDimensionNumbers(
    offset_dims=(), collapsed_slice_dims=(0,), start_index_map=(0,))


def _bcast_last(v, idx15):
    return lax.gather(v, idx15[:, None], _GATHER_DNUMS, slice_sizes=(1,),
                      mode=lax.GatherScatterMode.PROMISE_IN_BOUNDS)


def _row_layernorm(rows_v, out_v, r, cst, gs, bs, idx15, hidden):
    nsub = hidden // 16
    xs = []
    for j in range(nsub):
        xs.append(rows_v[r, pl.ds(j * 16, 16)] + cst[j])
    s = xs[0]
    q = xs[0] * xs[0]
    for j in range(1, nsub):
        s = s + xs[j]
        q = q + xs[j] * xs[j]
    ssum = _bcast_last(jnp.cumsum(s), idx15)
    qsum = _bcast_last(jnp.cumsum(q), idx15)
    inv_h = 1.0 / hidden
    mean = ssum * inv_h
    var = qsum * inv_h - mean * mean + EPS
    i = lax.bitcast_convert_type(var, jnp.int32)
    i = jnp.int32(0x5F3759DF) - lax.shift_right_logical(i, 1)
    y = lax.bitcast_convert_type(i, jnp.float32)
    half_v = 0.5 * var
    for _ in range(3):
        y = y * (1.5 - half_v * y * y)
    for j in range(nsub):
        out_v[r, pl.ds(j * 16, 16)] = (xs[j] - mean) * y * gs[j] + bs[j]


def _make_sc_kernel(B, L, V, H):
    T = B * L
    per_w = T // NW
    n_chunks = per_w // CHUNK
    chunks_per_col = B // CHUNK
    n_groups = n_chunks // NBUF
    assert n_chunks % NBUF == 0 and per_w % CHUNK == 0
    mesh = plsc.VectorSubcoreMesh(core_axis_name="c", subcore_axis_name="s")
    cp = pltpu.CompilerParams()
    if "needs_layout_passes" in pltpu.CompilerParams.__dataclass_fields__:
        cp = dataclasses.replace(cp, needs_layout_passes=False)

    @functools.partial(
        pl.kernel,
        mesh=mesh,
        compiler_params=cp,
        out_type=jax.ShapeDtypeStruct((B, L, H), jnp.float32),
        scratch_types=(
            [
                pltpu.VMEM((POS_ROWS, H), jnp.float32),
                pltpu.VMEM((H,), jnp.float32),
                pltpu.VMEM((H,), jnp.float32),
                pltpu.VMEM((H,), jnp.float32),
                pltpu.VMEM((per_w,), jnp.int32),
            ]
            + [pltpu.VMEM((CHUNK, H), jnp.float32) for _ in range(2 * NBUF)]
            + [pltpu.SemaphoreType.DMA for _ in range(2 * NBUF + 1)]
        ),
    )
    def k(ids_hbm, table_hbm, pos_hbm, tok_hbm, gamma_hbm, beta_hbm, out_hbm,
          *scratch):
        pos_v, tok_v, g_v, b_v, idx_v = scratch[:5]
        rows = scratch[5:5 + NBUF]
        outb = scratch[5 + NBUF:5 + 2 * NBUF]
        gsem = scratch[5 + 2 * NBUF:5 + 3 * NBUF]
        ssem = scratch[5 + 3 * NBUF:5 + 4 * NBUF]
        isem = scratch[5 + 4 * NBUF]

        wid = lax.axis_index("s") * NC + lax.axis_index("c")
        ci0 = wid * n_chunks
        l0 = lax.div(lax.div(ci0, chunks_per_col), 8) * 8
        nsub = H // 16

        idx_cp = pltpu.async_copy(
            ids_hbm.at[pl.ds(ci0 * CHUNK, per_w)], idx_v, isem)
        pltpu.sync_copy(pos_hbm.at[pl.ds(l0, POS_ROWS)], pos_v)
        pltpu.sync_copy(tok_hbm.at[0], tok_v)
        pltpu.sync_copy(gamma_hbm, g_v)
        pltpu.sync_copy(beta_hbm, b_v)

        @pl.loop(0, POS_ROWS)
        def _(r):
            for j in range(nsub):
                sl = pl.ds(j * 16, 16)
                pos_v[r, sl] = pos_v[r, sl] + tok_v[sl]

        gs = [g_v[pl.ds(j * 16, 16)] for j in range(nsub)]
        bs = [b_v[pl.ds(j * 16, 16)] for j in range(nsub)]
        idx15 = jnp.full((16,), 15, jnp.int32)

        idx_cp.wait()

        def gather_start(c, b):
            pltpu.async_copy(
                table_hbm.at[idx_v.at[pl.ds(c * CHUNK, CHUNK)]], rows[b],
                gsem[b])

        def gather_wait(c, b):
            pltpu.make_async_copy(
                table_hbm.at[idx_v.at[pl.ds(c * CHUNK, CHUNK)]], rows[b],
                gsem[b]).wait()

        def out_slice(c):
            ci = ci0 + c
            col = lax.div(ci, chunks_per_col)
            b0 = lax.rem(ci, chunks_per_col) * CHUNK
            return out_hbm.at[pl.ds(b0, CHUNK), col]

        def store_start(c, b):
            pltpu.async_copy(outb[b], out_slice(c), ssem[b])

        def store_wait(c, b):
            pltpu.make_async_copy(outb[b], out_slice(c), ssem[b]).wait()

        for b in range(LOOKAHEAD):
            gather_start(b, b)

        @pl.loop(0, n_groups)
        def _(g):
            for b in range(NBUF):
                c = g * NBUF + b
                ci = ci0 + c
                loff = lax.div(ci, chunks_per_col) - l0
                cst = [pos_v[loff, pl.ds(j * 16, 16)] for j in range(nsub)]

                gather_wait(c, b)

                @pl.when(c >= NBUF)
                def _():
                    store_wait(c - NBUF, b)

                @pl.loop(0, CHUNK, step=ROWS_PER_ITER)
                def _(r0):
                    for u in range(ROWS_PER_ITER):
                        _row_layernorm(rows[b], outb[b], r0 + u, cst, gs, bs,
                                       idx15, H)

                @pl.when(c + LOOKAHEAD < n_chunks)
                def _():
                    gather_start(c + LOOKAHEAD, b)

                store_start(c, b)

        for b in range(NBUF):
            store_wait(n_chunks - NBUF + b, b)

    return k


def kernel(input_ids, word_emb, pos_emb, tok_type_emb, gamma, beta):
    B, L = input_ids.shape
    V, H = word_emb.shape
    ids_t = input_ids.T.reshape(B * L).astype(jnp.int32)
    k = _make_sc_kernel(B, L, V, H)
    return k(ids_t, word_emb, pos_emb, tok_type_emb, gamma, beta)
